# Initial kernel scaffold; baseline (speedup 1.0000x reference)
import jax
import jax.numpy as jnp
from jax.experimental import pallas as pl


def kernel(x, edge_index, W1, a_src1, a_dst1, b1, W2, a_src2, a_dst2, b2, A, bA, C, bC):
    raise NotImplementedError("write your pallas kernel here")


# speedup vs baseline: 9.3202x; 9.3202x over previous
import jax
import jax.numpy as jnp
from jax import lax
from jax.experimental import pallas as pl
from jax.experimental.pallas import tpu as pltpu

_N = 14
_B = 32
_SEQ = 24
_R = _B * _N
_NEG = -1e30


def _dotT(a, b):
    return lax.dot_general(a, b, (((1,), (1,)), ((), ())),
                           preferred_element_type=jnp.float32)


def _dot(a, b):
    return lax.dot_general(a, b, (((1,), (0,)), ((), ())),
                           preferred_element_type=jnp.float32)


def _body(src_ref, dst_ref, xt_ref, W1_ref, as1_ref, ad1_ref, b1_ref,
          W2_ref, as2_ref, ad2_ref, b2_ref, Abig_ref, bA_ref, Cbig_ref,
          bC_ref, out_ref):
    f32 = jnp.float32

    iota_n_col = lax.broadcasted_iota(jnp.int32, (_N, 196), 0)
    iota_n_row = lax.broadcasted_iota(jnp.int32, (196, _N), 1)
    oh_dst = (iota_n_col == dst_ref[:]).astype(f32)
    oh_src = (iota_n_row == src_ref[:]).astype(f32)
    eye = (lax.broadcasted_iota(jnp.int32, (_N, _N), 0) ==
           lax.broadcasted_iota(jnp.int32, (_N, _N), 1)).astype(f32)
    cnt = _dot(oh_dst, oh_src) + eye

    r_col = lax.broadcasted_iota(jnp.int32, (_R, 1), 0)
    c_row = lax.broadcasted_iota(jnp.int32, (1, _R), 1)
    prow = ((r_col % _N) ==
            lax.broadcasted_iota(jnp.int32, (_R, _N), 1)).astype(f32)
    cnt_rows = _dot(prow, cnt)
    cnt_bd = _dotT(cnt_rows, prow)
    same_batch = (r_col // _N) == (c_row // _N)
    cnt_bd = jnp.where(same_batch, cnt_bd, 0.0)
    valid = cnt_bd > 0.0

    def gat_layer(h, W, a_s, a_d, b):
        hw = _dot(h, W)
        ad_col = _dotT(hw, a_d)
        as_row = _dotT(a_s, hw)
        alpha = ad_col + as_row
        alpha = jnp.where(alpha > 0.0, alpha, 0.2 * alpha)
        amax = jnp.max(jnp.where(valid, alpha, _NEG), axis=1, keepdims=True)
        e = jnp.where(valid, jnp.exp(alpha - amax), 0.0) * cnt_bd
        denom = jnp.sum(e, axis=1, keepdims=True) + 1e-16
        return _dot(e, hw) / denom + b

    out1 = gat_layer(xt_ref[:], W1_ref[:], as1_ref[:], ad1_ref[:], b1_ref[:])
    h1 = jnp.where(out1 > 0.0, out1, jnp.exp(jnp.minimum(out1, 0.0)) - 1.0)
    out2 = gat_layer(h1, W2_ref[:], as2_ref[:], ad2_ref[:], b2_ref[:])

    node = r_col % _N
    t1 = jnp.concatenate([out2] * _N, axis=1)
    k1 = lax.broadcasted_iota(jnp.int32, (1, _N * _SEQ), 1) // _SEQ
    x1 = jnp.where(node == k1, t1, 0.0)
    hmid = _dot(x1, Abig_ref[:]) + _dot(prow, bA_ref[:])
    hmid = jnp.maximum(hmid, 0.0)

    t2 = jnp.concatenate([hmid] * _N, axis=1)
    k2 = lax.broadcasted_iota(jnp.int32, (1, _N * 64), 1) // 64
    x2 = jnp.where(node == k2, t2, 0.0)
    out_ref[:] = _dot(x2, Cbig_ref[:]) + _dot(prow, bC_ref[:])


def kernel(x, edge_index, W1, a_src1, a_dst1, b1, W2, a_src2, a_dst2, b2,
           A, bA, C, bC):
    xt = jnp.transpose(x, (0, 2, 1)).reshape(_R, _SEQ)
    src = edge_index[0].reshape(196, 1)
    dst = edge_index[1].reshape(1, 196)
    pred = pl.pallas_call(
        _body,
        out_shape=jax.ShapeDtypeStruct((_R, _SEQ), jnp.float32),
    )(src, dst, xt,
      W1, a_src1.reshape(1, 64), a_dst1.reshape(1, 64), b1.reshape(1, 64),
      W2, a_src2.reshape(1, 24), a_dst2.reshape(1, 24), b2.reshape(1, 24),
      A.reshape(_N * _SEQ, 64), bA, C.reshape(_N * 64, _SEQ), bC)
    return jnp.transpose(pred.reshape(_B, _N, _SEQ), (1, 0, 2))


# speedup vs baseline: 10.2687x; 1.1018x over previous
import jax
import jax.numpy as jnp
from jax import lax
from jax.experimental import pallas as pl

_N = 14
_B = 32
_SEQ = 24
_R = _B * _N
_NEG = -1e30


def _dotT(a, b):
    return lax.dot_general(a, b, (((1,), (1,)), ((), ())),
                           preferred_element_type=jnp.float32)


def _dot(a, b):
    return lax.dot_general(a, b, (((1,), (0,)), ((), ())),
                           preferred_element_type=jnp.float32)


def _body(src_ref, dst_ref, xt_ref, W1_ref, as1_ref, ad1_ref, b1_ref,
          W2_ref, as2_ref, ad2_ref, b2_ref, Abig_ref, bA_ref, Cbig_ref,
          bC_ref, out_ref):
    f32 = jnp.float32

    iota_n_col = lax.broadcasted_iota(jnp.int32, (_N, 196), 0)
    iota_n_row = lax.broadcasted_iota(jnp.int32, (196, _N), 1)
    oh_dst = (iota_n_col == dst_ref[:]).astype(f32)
    oh_src = (iota_n_row == src_ref[:]).astype(f32)
    eye = (lax.broadcasted_iota(jnp.int32, (_N, _N), 0) ==
           lax.broadcasted_iota(jnp.int32, (_N, _N), 1)).astype(f32)
    cnt = _dot(oh_dst, oh_src) + eye

    r_col = lax.broadcasted_iota(jnp.int32, (_R, 1), 0)
    c_row = lax.broadcasted_iota(jnp.int32, (1, _R), 1)
    node = r_col // _B
    prow = (node ==
            lax.broadcasted_iota(jnp.int32, (_R, _N), 1)).astype(f32)
    cnt_rows = _dot(prow, cnt)
    valid = cnt_rows > 0.0
    sb = ((r_col % _B) == (c_row % _B)).astype(f32)

    def gat_layer(h, W, a_s, a_d, b):
        hw = _dot(h, W)
        ad_col = _dotT(hw, a_d)
        as_col = _dotT(hw, a_s)
        as_rep = _dot(sb, prow * as_col)
        alpha = ad_col + as_rep
        alpha = jnp.where(alpha > 0.0, alpha, 0.2 * alpha)
        amax = jnp.max(jnp.where(valid, alpha, _NEG), axis=1, keepdims=True)
        e = jnp.where(valid, jnp.exp(alpha - amax), 0.0) * cnt_rows
        denom = jnp.sum(e, axis=1, keepdims=True) + 1e-16
        e_bd = _dotT(e, prow) * sb
        return _dot(e_bd, hw) / denom + b

    out1 = gat_layer(xt_ref[:], W1_ref[:], as1_ref[:], ad1_ref[:], b1_ref[:])
    h1 = jnp.where(out1 > 0.0, out1, jnp.exp(jnp.minimum(out1, 0.0)) - 1.0)
    out2 = gat_layer(h1, W2_ref[:], as2_ref[:], ad2_ref[:], b2_ref[:])

    t1 = jnp.concatenate([out2] * _N, axis=1)
    k1 = lax.broadcasted_iota(jnp.int32, (1, _N * _SEQ), 1) // _SEQ
    x1 = jnp.where(node == k1, t1, 0.0)
    hmid = _dot(x1, Abig_ref[:]) + _dot(prow, bA_ref[:])
    hmid = jnp.maximum(hmid, 0.0)

    t2 = jnp.concatenate([hmid] * _N, axis=1)
    k2 = lax.broadcasted_iota(jnp.int32, (1, _N * 64), 1) // 64
    x2 = jnp.where(node == k2, t2, 0.0)
    out_ref[:] = _dot(x2, Cbig_ref[:]) + _dot(prow, bC_ref[:])


def kernel(x, edge_index, W1, a_src1, a_dst1, b1, W2, a_src2, a_dst2, b2,
           A, bA, C, bC):
    xt = jnp.transpose(x, (2, 0, 1)).reshape(_R, _SEQ)
    src = edge_index[0].reshape(196, 1)
    dst = edge_index[1].reshape(1, 196)
    pred = pl.pallas_call(
        _body,
        out_shape=jax.ShapeDtypeStruct((_R, _SEQ), jnp.float32),
    )(src, dst, xt,
      W1, a_src1.reshape(1, 64), a_dst1.reshape(1, 64), b1.reshape(1, 64),
      W2, a_src2.reshape(1, 24), a_dst2.reshape(1, 24), b2.reshape(1, 24),
      A.reshape(_N * _SEQ, 64), bA, C.reshape(_N * 64, _SEQ), bC)
    return pred.reshape(_N, _B, _SEQ)


# speedup vs baseline: 10.6797x; 1.0400x over previous
import jax
import jax.numpy as jnp
from jax import lax
from jax.experimental import pallas as pl

_N = 14
_B = 32
_SEQ = 24
_E = 196
_R = _B * _N
_NEG = -1e30


def _dotT(a, b):
    return lax.dot_general(a, b, (((1,), (1,)), ((), ())),
                           preferred_element_type=jnp.float32)


def _dot0(a, b):
    return lax.dot_general(a, b, (((0,), (0,)), ((), ())),
                           preferred_element_type=jnp.float32)


def _dot(a, b):
    return lax.dot_general(a, b, (((1,), (0,)), ((), ())),
                           preferred_element_type=jnp.float32)


def _body(x_ref, ei_ref, W1_ref, as1_ref, ad1_ref, b1_ref,
          W2_ref, as2_ref, ad2_ref, b2_ref, A_ref, bA_ref, C_ref,
          bC_ref, out_ref):
    f32 = jnp.float32

    iota_ne = lax.broadcasted_iota(jnp.int32, (_N, _E), 0)
    oh_srcT = (iota_ne == ei_ref[0:1, :]).astype(f32)
    oh_dst = (iota_ne == ei_ref[1:2, :]).astype(f32)
    eye = (lax.broadcasted_iota(jnp.int32, (_N, _N), 0) ==
           lax.broadcasted_iota(jnp.int32, (_N, _N), 1)).astype(f32)
    cnt = _dotT(oh_dst, oh_srcT) + eye

    r_col = lax.broadcasted_iota(jnp.int32, (_R, 1), 0)
    c_row = lax.broadcasted_iota(jnp.int32, (1, _R), 1)
    node = r_col % _N
    prow = (node ==
            lax.broadcasted_iota(jnp.int32, (_R, _N), 1)).astype(f32)
    cnt_rows = _dot(prow, cnt)
    valid = cnt_rows > 0.0
    sb = ((r_col // _N) == (c_row // _N)).astype(f32)

    def gat_attend(hw, a_s, a_d, b):
        ad_col = _dotT(hw, a_d)
        as_col = _dotT(hw, a_s)
        as_rep = _dot(sb, prow * as_col)
        alpha = ad_col + as_rep
        alpha = jnp.where(alpha > 0.0, alpha, 0.2 * alpha)
        amax = jnp.max(jnp.where(valid, alpha, _NEG), axis=1, keepdims=True)
        e = jnp.where(valid, jnp.exp(alpha - amax), 0.0) * cnt_rows
        denom = jnp.sum(e, axis=1, keepdims=True) + 1e-16
        e_bd = _dotT(e, prow) * sb
        return _dot(e_bd, hw) / denom + b

    V = jnp.concatenate([x_ref[b] for b in range(_B)], axis=1)
    hw1 = _dot0(V, W1_ref[:])
    out1 = gat_attend(hw1, as1_ref[:].reshape(1, 64),
                      ad1_ref[:].reshape(1, 64), b1_ref[:].reshape(1, 64))
    h1 = jnp.where(out1 > 0.0, out1, jnp.exp(jnp.minimum(out1, 0.0)) - 1.0)

    hw2 = _dot(h1, W2_ref[:])
    out2 = gat_attend(hw2, as2_ref[:].reshape(1, 24),
                      ad2_ref[:].reshape(1, 24), b2_ref[:].reshape(1, 24))

    t1 = jnp.concatenate([out2] * _N, axis=1)
    k1 = lax.broadcasted_iota(jnp.int32, (1, _N * _SEQ), 1) // _SEQ
    x1 = jnp.where(node == k1, t1, 0.0)
    hmid = _dot(x1, A_ref[:].reshape(_N * _SEQ, 64)) + _dot(prow, bA_ref[:])
    hmid = jnp.maximum(hmid, 0.0)

    t2 = jnp.concatenate([hmid] * _N, axis=1)
    k2 = lax.broadcasted_iota(jnp.int32, (1, _N * 64), 1) // 64
    x2 = jnp.where(node == k2, t2, 0.0)
    pred = _dot(x2, C_ref[:].reshape(_N * 64, _SEQ)) + _dot(prow, bC_ref[:])

    perm = (c_row == ((r_col % _B) * _N + r_col // _B)).astype(f32)
    out_ref[:] = _dot(perm, pred).reshape(_N, _B, _SEQ)


def kernel(x, edge_index, W1, a_src1, a_dst1, b1, W2, a_src2, a_dst2, b2,
           A, bA, C, bC):
    return pl.pallas_call(
        _body,
        out_shape=jax.ShapeDtypeStruct((_N, _B, _SEQ), jnp.float32),
    )(x, edge_index, W1, a_src1, a_dst1, b1, W2, a_src2, a_dst2, b2,
      A, bA, C, bC)


# speedup vs baseline: 23.8291x; 2.2313x over previous
import jax
import jax.numpy as jnp
from jax import lax
from jax.experimental import pallas as pl

_N = 14
_B = 32
_SEQ = 24
_E = 196
_R = _B * _N
_NEG = -1e30


def _dotT(a, b):
    return lax.dot_general(a, b, (((1,), (1,)), ((), ())),
                           preferred_element_type=jnp.float32)


def _dot0(a, b):
    return lax.dot_general(a, b, (((0,), (0,)), ((), ())),
                           preferred_element_type=jnp.float32)


def _dot(a, b):
    return lax.dot_general(a, b, (((1,), (0,)), ((), ())),
                           preferred_element_type=jnp.float32)


def _body(xl_ref, ei_ref, W1_ref, as1_ref, ad1_ref, b1_ref,
          W2T_ref, as2_ref, ad2_ref, b2_ref, A_ref, bA_ref, CT_ref,
          bC_ref, out_ref):
    f32 = jnp.float32

    iota_ne = lax.broadcasted_iota(jnp.int32, (_N, _E), 0)
    oh_srcT = (iota_ne == ei_ref[0:1, :]).astype(f32)
    oh_dst = (iota_ne == ei_ref[1:2, :]).astype(f32)
    eye = (lax.broadcasted_iota(jnp.int32, (_N, _N), 0) ==
           lax.broadcasted_iota(jnp.int32, (_N, _N), 1)).astype(f32)
    cnt = _dotT(oh_dst, oh_srcT) + eye

    r_col = lax.broadcasted_iota(jnp.int32, (_R, 1), 0)
    c_row = lax.broadcasted_iota(jnp.int32, (1, _R), 1)
    node = r_col // _B
    prow = (node ==
            lax.broadcasted_iota(jnp.int32, (_R, _N), 1)).astype(f32)
    cnt_rows = _dot(prow, cnt)
    valid = cnt_rows > 0.0
    sb = ((r_col % _B) == (c_row % _B)).astype(f32)

    def gat_attend(hw, a_s, a_d, b):
        ad_col = _dotT(hw, a_d)
        as_col = _dotT(hw, a_s)
        as_rep = _dot(sb, prow * as_col)
        alpha = ad_col + as_rep
        alpha = jnp.where(alpha > 0.0, alpha, 0.2 * alpha)
        amax = jnp.max(jnp.where(valid, alpha, _NEG), axis=1, keepdims=True)
        e = jnp.where(valid, jnp.exp(alpha - amax), 0.0) * cnt_rows
        denom = jnp.sum(e, axis=1, keepdims=True) + 1e-16
        e_bd = _dotT(e, prow) * sb
        return _dot(e_bd, hw) / denom + b

    V = jnp.concatenate([xl_ref[d] for d in range(_N)], axis=1)
    hw1 = _dot0(V, W1_ref[:])
    out1 = gat_attend(hw1, as1_ref[:].reshape(1, 64),
                      ad1_ref[:].reshape(1, 64), b1_ref[:].reshape(1, 64))
    h1 = jnp.where(out1 > 0.0, out1, jnp.exp(jnp.minimum(out1, 0.0)) - 1.0)

    hw2 = _dotT(h1, W2T_ref[:])
    out2 = gat_attend(hw2, as2_ref[:].reshape(1, 24),
                      ad2_ref[:].reshape(1, 24), b2_ref[:].reshape(1, 24))

    t1 = jnp.concatenate([out2] * _N, axis=1)
    k1 = lax.broadcasted_iota(jnp.int32, (1, _N * _SEQ), 1) // _SEQ
    x1 = jnp.where(node == k1, t1, 0.0)
    hmid = _dot(x1, A_ref[:].reshape(_N * _SEQ, 64)) + _dot(prow, bA_ref[:])
    hmid = jnp.maximum(hmid, 0.0)

    t2 = jnp.concatenate([hmid] * _N, axis=1)
    k2 = lax.broadcasted_iota(jnp.int32, (1, _N * 64), 1) // 64
    x2 = jnp.where(node == k2, t2, 0.0)
    CbigT = jnp.concatenate([CT_ref[k] for k in range(_N)], axis=1)
    pred = _dotT(x2, CbigT) + _dot(prow, bC_ref[:])

    i448 = (c_row == r_col).astype(f32)
    predT = _dot0(pred, i448)
    M = jnp.concatenate([predT[:, d * _B:(d + 1) * _B] for d in range(_N)],
                        axis=0)
    out_ref[:] = M.reshape(_N, _SEQ, _B)


def kernel(x, edge_index, W1, a_src1, a_dst1, b1, W2, a_src2, a_dst2, b2,
           A, bA, C, bC):
    xl = jnp.transpose(x, (2, 1, 0))
    W2T = jnp.transpose(W2)
    CT = jnp.transpose(C, (0, 2, 1))
    Z = pl.pallas_call(
        _body,
        out_shape=jax.ShapeDtypeStruct((_N, _SEQ, _B), jnp.float32),
    )(xl, edge_index, W1, a_src1, a_dst1, b1, W2T, a_src2, a_dst2, b2,
      A, bA, CT, bC)
    return jnp.transpose(Z, (0, 2, 1))
